# Initial kernel scaffold; baseline (speedup 1.0000x reference)
import jax
import jax.numpy as jnp
from jax.experimental import pallas as pl


def kernel(sampleT, sampleU, sampleI, sampleJ, samplePR, sampleR, alpha, betaU, betaI, betaT, thetaU, gammaUI, gammaIU, gammaIJ, gammaJI, gammaTI, gammaIT):
    raise NotImplementedError("write your pallas kernel here")


# speedup vs baseline: 13.1248x; 13.1248x over previous
import functools

import jax
import jax.numpy as jnp
from jax import lax
from jax.experimental import pallas as pl
from jax.experimental.pallas import tpu as pltpu
from jax.experimental.pallas import tpu_sc as plsc

NC, NS = 2, 16
NW = NC * NS
NB = 16384
BPW = NB // NW
NGRP = BPW // 16
NCHUNK = BPW // 128
N_UI = 1000
N_T = 100
TIW = 128


def _tables_body(ui_ref, iu_ref, ij_ref, ji_ref, ti_ref, it_ref,
                 mui_ref, mij_ref, mti_ref):
    dn = (((1,), (1,)), ((), ()))
    mui_ref[...] = lax.dot_general(ui_ref[...], iu_ref[...], dn,
                                   preferred_element_type=jnp.float32)
    mij_ref[...] = lax.dot_general(ij_ref[...], ji_ref[...], dn,
                                   preferred_element_type=jnp.float32)
    mti_ref[...] = lax.dot_general(ti_ref[...], it_ref[...], dn,
                                   preferred_element_type=jnp.float32)


_tables = pl.pallas_call(
    _tables_body,
    out_shape=[
        jax.ShapeDtypeStruct((N_UI, N_UI), jnp.float32),
        jax.ShapeDtypeStruct((N_UI, N_UI), jnp.float32),
        jax.ShapeDtypeStruct((N_T, TIW), jnp.float32),
    ],
)

_mesh = plsc.VectorSubcoreMesh(core_axis_name="c", subcore_axis_name="s",
                               num_cores=NC, num_subcores=NS)


@functools.partial(
    pl.kernel,
    out_type=jax.ShapeDtypeStruct((NW, 16), jnp.float32),
    mesh=_mesh,
    compiler_params=pltpu.CompilerParams(needs_layout_passes=False),
    scratch_types=[
        pltpu.VMEM((BPW,), jnp.int32),
        pltpu.VMEM((BPW,), jnp.int32),
        pltpu.VMEM((BPW,), jnp.int32),
        pltpu.VMEM((BPW,), jnp.int32),
        pltpu.VMEM((BPW,), jnp.float32),
        pltpu.VMEM((BPW,), jnp.float32),
        pltpu.VMEM((16,), jnp.float32),
        pltpu.VMEM((1024,), jnp.float32),
        pltpu.VMEM((1024,), jnp.float32),
        pltpu.VMEM((128,), jnp.float32),
        pltpu.VMEM((1024,), jnp.float32),
        pltpu.VMEM((NCHUNK, 128), jnp.int32),
        pltpu.VMEM((NCHUNK, 128), jnp.int32),
        pltpu.VMEM((NCHUNK, 128), jnp.int32),
        pltpu.VMEM((NCHUNK, 128), jnp.float32),
        pltpu.VMEM((NCHUNK, 128), jnp.float32),
        pltpu.VMEM((NCHUNK, 128), jnp.float32),
        pltpu.VMEM((16,), jnp.float32),
        pltpu.SemaphoreType.DMA,
    ],
)
def _sc_loss(t_hbm, u_hbm, i_hbm, j_hbm, pr_hbm, r_hbm, a_hbm,
             bu_hbm, bi_hbm, bt_hbm, th_hbm,
             mui_hbm, mij_hbm, mti_hbm, out_hbm,
             t_v, u_v, i_v, j_v, pr_v, r_v, a_v,
             bu_v, bi_v, bt_v, th_v,
             iui_v, iij_v, iti_v, gui_v, gij_v, gti_v, acc_v, sem):
    wid = lax.axis_index("s") * NC + lax.axis_index("c")
    base = wid * BPW

    pltpu.sync_copy(t_hbm.at[pl.ds(base, BPW)], t_v)
    pltpu.sync_copy(u_hbm.at[pl.ds(base, BPW)], u_v)
    pltpu.sync_copy(i_hbm.at[pl.ds(base, BPW)], i_v)
    pltpu.sync_copy(j_hbm.at[pl.ds(base, BPW)], j_v)
    pltpu.sync_copy(pr_hbm.at[pl.ds(base, BPW)], pr_v)
    pltpu.sync_copy(r_hbm.at[pl.ds(base, BPW)], r_v)
    pltpu.sync_copy(a_hbm, a_v)
    pltpu.sync_copy(bu_hbm, bu_v.at[pl.ds(0, N_UI)])
    pltpu.sync_copy(bi_hbm, bi_v.at[pl.ds(0, N_UI)])
    pltpu.sync_copy(bt_hbm, bt_v.at[pl.ds(0, N_T)])
    pltpu.sync_copy(th_hbm, th_v.at[pl.ds(0, N_UI)])

    for g in range(NGRP):
        sl = pl.ds(g * 16, 16)
        c, o = g // 8, (g % 8) * 16
        osl = pl.ds(o, 16)
        tt = t_v[sl]
        uu = u_v[sl]
        ii = i_v[sl]
        jj = j_v[sl]
        iui_v[c, osl] = uu * N_UI + ii
        iij_v[c, osl] = ii * N_UI + jj
        iti_v[c, osl] = tt * TIW + jnp.minimum(ii, N_T)

    copies = []
    for c in range(NCHUNK):
        copies.append(pltpu.async_copy(mui_hbm.at[iui_v.at[c]], gui_v.at[c], sem))
        copies.append(pltpu.async_copy(mij_hbm.at[iij_v.at[c]], gij_v.at[c], sem))
        copies.append(pltpu.async_copy(mti_hbm.at[iti_v.at[c]], gti_v.at[c], sem))
    for cp in copies:
        cp.wait()

    acc = jnp.zeros((16,), jnp.float32)
    for g in range(NGRP):
        sl = pl.ds(g * 16, 16)
        c, o = g // 8, (g % 8) * 16
        osl = pl.ds(o, 16)
        tt = t_v[sl]
        uu = u_v[sl]
        ii = i_v[sl]
        bias = (plsc.load_gather(bi_v, [ii])
                + plsc.load_gather(bt_v, [tt])
                + plsc.load_gather(th_v, [uu]) * pr_v[sl]
                + plsc.load_gather(bu_v, [uu])
                + gui_v[c, osl] + gij_v[c, osl] + gti_v[c, osl])
        diff = a_v[:] + bias - r_v[sl]
        acc = acc + diff * diff
    acc_v[:] = acc
    pltpu.sync_copy(acc_v, out_hbm.at[wid])


def kernel(sampleT, sampleU, sampleI, sampleJ, samplePR, sampleR, alpha,
           betaU, betaI, betaT, thetaU,
           gammaUI, gammaIU, gammaIJ, gammaJI, gammaTI, gammaIT):
    it_pad = jnp.pad(gammaIT, ((0, TIW - N_T), (0, 0)),
                     constant_values=jnp.nan)
    mui, mij, mti = _tables(gammaUI, gammaIU, gammaIJ, gammaJI,
                            gammaTI, it_pad)
    alpha16 = jnp.full((16,), alpha, jnp.float32)
    out = _sc_loss(sampleT, sampleU, sampleI, sampleJ, samplePR, sampleR,
                   alpha16, betaU, betaI, betaT, thetaU,
                   mui.reshape(-1), mij.reshape(-1), mti.reshape(-1))
    return jnp.sum(out) * (0.5 / NB)


# speedup vs baseline: 17.4448x; 1.3291x over previous
import functools

import jax
import jax.numpy as jnp
from jax import lax
from jax.experimental import pallas as pl
from jax.experimental.pallas import tpu as pltpu
from jax.experimental.pallas import tpu_sc as plsc

NC, NS = 2, 16
NW = NC * NS
NB = 16384
BPW = NB // NW
NGRP = BPW // 16
NCHUNK = BPW // 128
N_UI = 1000
N_T = 100
TIW = 128


def _tables_body(ui_ref, iu_ref, ij_ref, ji_ref, ti_ref, it_ref,
                 mui_ref, mij_ref, mti_ref):
    dn = (((0,), (0,)), ((), ()))
    mui_ref[...] = lax.dot_general(ui_ref[...], iu_ref[...], dn,
                                   preferred_element_type=jnp.float32)
    mij_ref[...] = lax.dot_general(ij_ref[...], ji_ref[...], dn,
                                   preferred_element_type=jnp.float32)
    mti_ref[...] = lax.dot_general(ti_ref[...], it_ref[...], dn,
                                   preferred_element_type=jnp.float32)


_tables = pl.pallas_call(
    _tables_body,
    out_shape=[
        jax.ShapeDtypeStruct((N_UI, N_UI), jnp.float32),
        jax.ShapeDtypeStruct((N_UI, N_UI), jnp.float32),
        jax.ShapeDtypeStruct((N_T, TIW), jnp.float32),
    ],
)

_mesh = plsc.VectorSubcoreMesh(core_axis_name="c", subcore_axis_name="s",
                               num_cores=NC, num_subcores=NS)


@functools.partial(
    pl.kernel,
    out_type=jax.ShapeDtypeStruct((NW, 16), jnp.float32),
    mesh=_mesh,
    compiler_params=pltpu.CompilerParams(needs_layout_passes=False),
    scratch_types=[
        pltpu.VMEM((BPW,), jnp.int32),
        pltpu.VMEM((BPW,), jnp.int32),
        pltpu.VMEM((BPW,), jnp.int32),
        pltpu.VMEM((BPW,), jnp.int32),
        pltpu.VMEM((BPW,), jnp.float32),
        pltpu.VMEM((BPW,), jnp.float32),
        pltpu.VMEM((16,), jnp.float32),
        pltpu.VMEM((1024,), jnp.float32),
        pltpu.VMEM((1024,), jnp.float32),
        pltpu.VMEM((128,), jnp.float32),
        pltpu.VMEM((1024,), jnp.float32),
        pltpu.VMEM((NCHUNK, 128), jnp.int32),
        pltpu.VMEM((NCHUNK, 128), jnp.int32),
        pltpu.VMEM((NCHUNK, 128), jnp.int32),
        pltpu.VMEM((NCHUNK, 128), jnp.float32),
        pltpu.VMEM((NCHUNK, 128), jnp.float32),
        pltpu.VMEM((NCHUNK, 128), jnp.float32),
        pltpu.VMEM((BPW,), jnp.float32),
        pltpu.VMEM((16,), jnp.float32),
        pltpu.SemaphoreType.DMA,
        pltpu.SemaphoreType.DMA,
    ],
)
def _sc_loss(t_hbm, u_hbm, i_hbm, j_hbm, pr_hbm, r_hbm, a_hbm,
             bu_hbm, bi_hbm, bt_hbm, th_hbm,
             mui_hbm, mij_hbm, mti_hbm, out_hbm,
             t_v, u_v, i_v, j_v, pr_v, r_v, a_v,
             bu_v, bi_v, bt_v, th_v,
             iui_v, iij_v, iti_v, gui_v, gij_v, gti_v, bias_v, acc_v,
             sem, gsem):
    wid = lax.axis_index("s") * NC + lax.axis_index("c")
    base = wid * BPW

    stage = [
        pltpu.async_copy(t_hbm.at[pl.ds(base, BPW)], t_v, sem),
        pltpu.async_copy(u_hbm.at[pl.ds(base, BPW)], u_v, sem),
        pltpu.async_copy(i_hbm.at[pl.ds(base, BPW)], i_v, sem),
        pltpu.async_copy(j_hbm.at[pl.ds(base, BPW)], j_v, sem),
        pltpu.async_copy(pr_hbm.at[pl.ds(base, BPW)], pr_v, sem),
        pltpu.async_copy(r_hbm.at[pl.ds(base, BPW)], r_v, sem),
        pltpu.async_copy(a_hbm, a_v, sem),
        pltpu.async_copy(bu_hbm, bu_v.at[pl.ds(0, N_UI)], sem),
        pltpu.async_copy(bi_hbm, bi_v.at[pl.ds(0, N_UI)], sem),
        pltpu.async_copy(bt_hbm, bt_v.at[pl.ds(0, N_T)], sem),
        pltpu.async_copy(th_hbm, th_v.at[pl.ds(0, N_UI)], sem),
    ]
    for cp in stage:
        cp.wait()

    for g in range(NGRP):
        sl = pl.ds(g * 16, 16)
        c, o = g // 8, (g % 8) * 16
        osl = pl.ds(o, 16)
        tt = t_v[sl]
        uu = u_v[sl]
        ii = i_v[sl]
        jj = j_v[sl]
        iui_v[c, osl] = uu * N_UI + ii
        iij_v[c, osl] = ii * N_UI + jj
        iti_v[c, osl] = tt * TIW + jnp.minimum(ii, N_T)

    copies = []
    for c in range(NCHUNK):
        copies.append(pltpu.async_copy(mui_hbm.at[iui_v.at[c]], gui_v.at[c], gsem))
        copies.append(pltpu.async_copy(mij_hbm.at[iij_v.at[c]], gij_v.at[c], gsem))
        copies.append(pltpu.async_copy(mti_hbm.at[iti_v.at[c]], gti_v.at[c], gsem))

    for g in range(NGRP):
        sl = pl.ds(g * 16, 16)
        tt = t_v[sl]
        uu = u_v[sl]
        ii = i_v[sl]
        bias_v[sl] = (a_v[:] - r_v[sl]
                      + plsc.load_gather(bi_v, [ii])
                      + plsc.load_gather(bt_v, [tt])
                      + plsc.load_gather(th_v, [uu]) * pr_v[sl]
                      + plsc.load_gather(bu_v, [uu]))

    for cp in copies:
        cp.wait()

    acc = jnp.zeros((16,), jnp.float32)
    for g in range(NGRP):
        sl = pl.ds(g * 16, 16)
        c, o = g // 8, (g % 8) * 16
        osl = pl.ds(o, 16)
        diff = bias_v[sl] + gui_v[c, osl] + gij_v[c, osl] + gti_v[c, osl]
        acc = acc + diff * diff
    acc_v[:] = acc
    pltpu.sync_copy(acc_v, out_hbm.at[wid])


def kernel(sampleT, sampleU, sampleI, sampleJ, samplePR, sampleR, alpha,
           betaU, betaI, betaT, thetaU,
           gammaUI, gammaIU, gammaIJ, gammaJI, gammaTI, gammaIT):
    it_pad = jnp.pad(gammaIT.T, ((0, 0), (0, TIW - N_T)),
                     constant_values=jnp.nan)
    mui, mij, mti = _tables(gammaUI.T, gammaIU.T, gammaIJ.T, gammaJI.T,
                            gammaTI.T, it_pad)
    alpha16 = jnp.full((16,), alpha, jnp.float32)
    out = _sc_loss(sampleT, sampleU, sampleI, sampleJ, samplePR, sampleR,
                   alpha16, betaU, betaI, betaT, thetaU,
                   mui.reshape(-1), mij.reshape(-1), mti.reshape(-1))
    return jnp.sum(out) * (0.5 / NB)


# speedup vs baseline: 21.5040x; 1.2327x over previous
import functools

import jax
import jax.numpy as jnp
from jax import lax
from jax.experimental import pallas as pl
from jax.experimental.pallas import tpu as pltpu
from jax.experimental.pallas import tpu_sc as plsc

NC, NS = 2, 16
NW = NC * NS
NB = 16384
BPW = NB // NW
NGRP = BPW // 16
NCHUNK = BPW // 128
N_UI = 1000
N_T = 100
TIW = 128


def _tables_body(ui_ref, iu_ref, ij_ref, ji_ref, ti_ref, it_ref,
                 mui_ref, mij_ref, mti_ref):
    dn = (((0,), (0,)), ((), ()))
    mui_ref[:, 0, 0, :] = lax.dot_general(ui_ref[...], iu_ref[...], dn,
                                          preferred_element_type=jnp.float32)
    mij_ref[:, 0, 0, :] = lax.dot_general(ij_ref[...], ji_ref[...], dn,
                                          preferred_element_type=jnp.float32)

    @pl.when(pl.program_id(0) == 0)
    def _():
        it128 = jnp.concatenate(
            [it_ref[...], jnp.full((K, TIW - N_T), jnp.nan, jnp.float32)],
            axis=1)
        mti_ref[...] = lax.dot_general(ti_ref[...], it128, dn,
                                       preferred_element_type=jnp.float32)


K = 64
_tables = pl.pallas_call(
    _tables_body,
    grid=(8,),
    in_specs=[
        pl.BlockSpec((K, N_UI), lambda b: (0, 0)),
        pl.BlockSpec((K, 128), lambda b: (0, b)),
        pl.BlockSpec((K, N_UI), lambda b: (0, 0)),
        pl.BlockSpec((K, 128), lambda b: (0, b)),
        pl.BlockSpec((K, N_T), lambda b: (0, 0)),
        pl.BlockSpec((K, N_T), lambda b: (0, 0)),
    ],
    out_specs=[
        pl.BlockSpec((N_UI, 1, 1, 128), lambda b: (0, b, 0, 0)),
        pl.BlockSpec((N_UI, 1, 1, 128), lambda b: (0, b, 0, 0)),
        pl.BlockSpec((N_T, TIW), lambda b: (0, 0)),
    ],
    out_shape=[
        jax.ShapeDtypeStruct((N_UI, 8, 1, 128), jnp.float32),
        jax.ShapeDtypeStruct((N_UI, 8, 1, 128), jnp.float32),
        jax.ShapeDtypeStruct((N_T, TIW), jnp.float32),
    ],
)

_mesh = plsc.VectorSubcoreMesh(core_axis_name="c", subcore_axis_name="s",
                               num_cores=NC, num_subcores=NS)


@functools.partial(
    pl.kernel,
    out_type=jax.ShapeDtypeStruct((NW, 16), jnp.float32),
    mesh=_mesh,
    compiler_params=pltpu.CompilerParams(needs_layout_passes=False),
    scratch_types=[
        pltpu.VMEM((BPW,), jnp.int32),
        pltpu.VMEM((BPW,), jnp.int32),
        pltpu.VMEM((BPW,), jnp.int32),
        pltpu.VMEM((BPW,), jnp.int32),
        pltpu.VMEM((BPW,), jnp.float32),
        pltpu.VMEM((BPW,), jnp.float32),
        pltpu.VMEM((16,), jnp.float32),
        pltpu.VMEM((1024,), jnp.float32),
        pltpu.VMEM((1024,), jnp.float32),
        pltpu.VMEM((128,), jnp.float32),
        pltpu.VMEM((1024,), jnp.float32),
        pltpu.VMEM((NCHUNK, 128), jnp.int32),
        pltpu.VMEM((NCHUNK, 128), jnp.int32),
        pltpu.VMEM((NCHUNK, 128), jnp.int32),
        pltpu.VMEM((NCHUNK, 128), jnp.float32),
        pltpu.VMEM((NCHUNK, 128), jnp.float32),
        pltpu.VMEM((NCHUNK, 128), jnp.float32),
        pltpu.VMEM((BPW,), jnp.float32),
        pltpu.VMEM((16,), jnp.float32),
        pltpu.SemaphoreType.DMA,
        pltpu.SemaphoreType.DMA,
    ],
)
def _sc_loss(t_hbm, u_hbm, i_hbm, j_hbm, pr_hbm, r_hbm, a_hbm,
             bu_hbm, bi_hbm, bt_hbm, th_hbm,
             mui_hbm, mij_hbm, mti_hbm, out_hbm,
             t_v, u_v, i_v, j_v, pr_v, r_v, a_v,
             bu_v, bi_v, bt_v, th_v,
             iui_v, iij_v, iti_v, gui_v, gij_v, gti_v, bias_v, acc_v,
             sem, gsem):
    wid = lax.axis_index("s") * NC + lax.axis_index("c")
    base = wid * BPW

    stage = [
        pltpu.async_copy(t_hbm.at[pl.ds(base, BPW)], t_v, sem),
        pltpu.async_copy(u_hbm.at[pl.ds(base, BPW)], u_v, sem),
        pltpu.async_copy(i_hbm.at[pl.ds(base, BPW)], i_v, sem),
        pltpu.async_copy(j_hbm.at[pl.ds(base, BPW)], j_v, sem),
        pltpu.async_copy(pr_hbm.at[pl.ds(base, BPW)], pr_v, sem),
        pltpu.async_copy(r_hbm.at[pl.ds(base, BPW)], r_v, sem),
        pltpu.async_copy(a_hbm, a_v, sem),
        pltpu.async_copy(bu_hbm, bu_v.at[pl.ds(0, N_UI)], sem),
        pltpu.async_copy(bi_hbm, bi_v.at[pl.ds(0, N_UI)], sem),
        pltpu.async_copy(bt_hbm, bt_v.at[pl.ds(0, N_T)], sem),
        pltpu.async_copy(th_hbm, th_v.at[pl.ds(0, N_UI)], sem),
    ]
    for cp in stage:
        cp.wait()

    for g in range(NGRP):
        sl = pl.ds(g * 16, 16)
        c, o = g // 8, (g % 8) * 16
        osl = pl.ds(o, 16)
        tt = t_v[sl]
        uu = u_v[sl]
        ii = i_v[sl]
        jj = j_v[sl]
        iui_v[c, osl] = uu * 1024 + ii
        iij_v[c, osl] = ii * 1024 + jj
        iti_v[c, osl] = tt * TIW + jnp.minimum(ii, N_T)

    copies = []
    for c in range(NCHUNK):
        copies.append(pltpu.async_copy(mui_hbm.at[iui_v.at[c]], gui_v.at[c], gsem))
        copies.append(pltpu.async_copy(mij_hbm.at[iij_v.at[c]], gij_v.at[c], gsem))
        copies.append(pltpu.async_copy(mti_hbm.at[iti_v.at[c]], gti_v.at[c], gsem))

    for g in range(NGRP):
        sl = pl.ds(g * 16, 16)
        tt = t_v[sl]
        uu = u_v[sl]
        ii = i_v[sl]
        bias_v[sl] = (a_v[:] - r_v[sl]
                      + plsc.load_gather(bi_v, [ii])
                      + plsc.load_gather(bt_v, [tt])
                      + plsc.load_gather(th_v, [uu]) * pr_v[sl]
                      + plsc.load_gather(bu_v, [uu]))

    for cp in copies:
        cp.wait()

    acc = jnp.zeros((16,), jnp.float32)
    for g in range(NGRP):
        sl = pl.ds(g * 16, 16)
        c, o = g // 8, (g % 8) * 16
        osl = pl.ds(o, 16)
        diff = bias_v[sl] + gui_v[c, osl] + gij_v[c, osl] + gti_v[c, osl]
        acc = acc + diff * diff
    acc_v[:] = acc
    pltpu.sync_copy(acc_v, out_hbm.at[wid])


def kernel(sampleT, sampleU, sampleI, sampleJ, samplePR, sampleR, alpha,
           betaU, betaI, betaT, thetaU,
           gammaUI, gammaIU, gammaIJ, gammaJI, gammaTI, gammaIT):
    mui, mij, mti = _tables(gammaUI.T, gammaIU.T, gammaIJ.T, gammaJI.T,
                            gammaTI.T, gammaIT.T)
    alpha16 = jnp.full((16,), alpha, jnp.float32)
    out = _sc_loss(sampleT, sampleU, sampleI, sampleJ, samplePR, sampleR,
                   alpha16, betaU, betaI, betaT, thetaU,
                   mui.reshape(-1), mij.reshape(-1), mti.reshape(-1))
    return jnp.sum(out) * (0.5 / NB)


# speedup vs baseline: 22.4420x; 1.0436x over previous
import functools

import jax
import jax.numpy as jnp
from jax import lax
from jax.experimental import pallas as pl
from jax.experimental.pallas import tpu as pltpu
from jax.experimental.pallas import tpu_sc as plsc

NC, NS = 2, 16
NW = NC * NS
NB = 16384
BPW = NB // NW
NGRP = BPW // 16
NCHUNK = BPW // 128
N_UI = 1000
N_T = 100
TIW = 128


def _tables_body(ui_ref, iu_ref, ij_ref, ji_ref, ti_ref, it_ref,
                 mui_ref, mij_ref, mti_ref):
    dn = (((0,), (0,)), ((), ()))
    mui = lax.dot_general(ui_ref[...], iu_ref[...], dn,
                          preferred_element_type=jnp.float32)
    mij = lax.dot_general(ij_ref[...], ji_ref[...], dn,
                          preferred_element_type=jnp.float32)
    for k in range(4):
        mui_ref[:, k, 0, :] = mui[:, k * 128:(k + 1) * 128]
        mij_ref[:, k, 0, :] = mij[:, k * 128:(k + 1) * 128]

    @pl.when(pl.program_id(0) == 0)
    def _():
        it128 = jnp.concatenate(
            [it_ref[...], jnp.full((K, TIW - N_T), jnp.nan, jnp.float32)],
            axis=1)
        mti_ref[...] = lax.dot_general(ti_ref[...], it128, dn,
                                       preferred_element_type=jnp.float32)


K = 64
_tables = pl.pallas_call(
    _tables_body,
    grid=(2,),
    in_specs=[
        pl.BlockSpec((K, N_UI), lambda b: (0, 0)),
        pl.BlockSpec((K, 512), lambda b: (0, b)),
        pl.BlockSpec((K, N_UI), lambda b: (0, 0)),
        pl.BlockSpec((K, 512), lambda b: (0, b)),
        pl.BlockSpec((K, N_T), lambda b: (0, 0)),
        pl.BlockSpec((K, N_T), lambda b: (0, 0)),
    ],
    out_specs=[
        pl.BlockSpec((N_UI, 4, 1, 128), lambda b: (0, b, 0, 0)),
        pl.BlockSpec((N_UI, 4, 1, 128), lambda b: (0, b, 0, 0)),
        pl.BlockSpec((N_T, TIW), lambda b: (0, 0)),
    ],
    out_shape=[
        jax.ShapeDtypeStruct((N_UI, 8, 1, 128), jnp.float32),
        jax.ShapeDtypeStruct((N_UI, 8, 1, 128), jnp.float32),
        jax.ShapeDtypeStruct((N_T, TIW), jnp.float32),
    ],
)

_mesh = plsc.VectorSubcoreMesh(core_axis_name="c", subcore_axis_name="s",
                               num_cores=NC, num_subcores=NS)


@functools.partial(
    pl.kernel,
    out_type=jax.ShapeDtypeStruct((NW, 16), jnp.float32),
    mesh=_mesh,
    compiler_params=pltpu.CompilerParams(needs_layout_passes=False),
    scratch_types=[
        pltpu.VMEM((BPW,), jnp.int32),
        pltpu.VMEM((BPW,), jnp.int32),
        pltpu.VMEM((BPW,), jnp.int32),
        pltpu.VMEM((BPW,), jnp.int32),
        pltpu.VMEM((BPW,), jnp.float32),
        pltpu.VMEM((BPW,), jnp.float32),
        pltpu.VMEM((16,), jnp.float32),
        pltpu.VMEM((1024,), jnp.float32),
        pltpu.VMEM((1024,), jnp.float32),
        pltpu.VMEM((128,), jnp.float32),
        pltpu.VMEM((1024,), jnp.float32),
        pltpu.VMEM((NCHUNK, 128), jnp.int32),
        pltpu.VMEM((NCHUNK, 128), jnp.int32),
        pltpu.VMEM((NCHUNK, 128), jnp.int32),
        pltpu.VMEM((NCHUNK, 128), jnp.float32),
        pltpu.VMEM((NCHUNK, 128), jnp.float32),
        pltpu.VMEM((NCHUNK, 128), jnp.float32),
        pltpu.VMEM((BPW,), jnp.float32),
        pltpu.VMEM((16,), jnp.float32),
        pltpu.SemaphoreType.DMA,
        pltpu.SemaphoreType.DMA,
    ],
)
def _sc_loss(t_hbm, u_hbm, i_hbm, j_hbm, pr_hbm, r_hbm, a_hbm,
             bu_hbm, bi_hbm, bt_hbm, th_hbm,
             mui_hbm, mij_hbm, mti_hbm, out_hbm,
             t_v, u_v, i_v, j_v, pr_v, r_v, a_v,
             bu_v, bi_v, bt_v, th_v,
             iui_v, iij_v, iti_v, gui_v, gij_v, gti_v, bias_v, acc_v,
             sem, gsem):
    wid = lax.axis_index("s") * NC + lax.axis_index("c")
    base = wid * BPW

    stage = [
        pltpu.async_copy(t_hbm.at[pl.ds(base, BPW)], t_v, sem),
        pltpu.async_copy(u_hbm.at[pl.ds(base, BPW)], u_v, sem),
        pltpu.async_copy(i_hbm.at[pl.ds(base, BPW)], i_v, sem),
        pltpu.async_copy(j_hbm.at[pl.ds(base, BPW)], j_v, sem),
        pltpu.async_copy(pr_hbm.at[pl.ds(base, BPW)], pr_v, sem),
        pltpu.async_copy(r_hbm.at[pl.ds(base, BPW)], r_v, sem),
        pltpu.async_copy(a_hbm, a_v, sem),
        pltpu.async_copy(bu_hbm, bu_v.at[pl.ds(0, N_UI)], sem),
        pltpu.async_copy(bi_hbm, bi_v.at[pl.ds(0, N_UI)], sem),
        pltpu.async_copy(bt_hbm, bt_v.at[pl.ds(0, N_T)], sem),
        pltpu.async_copy(th_hbm, th_v.at[pl.ds(0, N_UI)], sem),
    ]
    for cp in stage:
        cp.wait()

    for g in range(NGRP):
        sl = pl.ds(g * 16, 16)
        c, o = g // 8, (g % 8) * 16
        osl = pl.ds(o, 16)
        tt = t_v[sl]
        uu = u_v[sl]
        ii = i_v[sl]
        jj = j_v[sl]
        iui_v[c, osl] = uu * 1024 + ii
        iij_v[c, osl] = ii * 1024 + jj
        iti_v[c, osl] = tt * TIW + jnp.minimum(ii, N_T)

    copies = []
    for c in range(NCHUNK):
        copies.append(pltpu.async_copy(mui_hbm.at[iui_v.at[c]], gui_v.at[c], gsem))
        copies.append(pltpu.async_copy(mij_hbm.at[iij_v.at[c]], gij_v.at[c], gsem))
        copies.append(pltpu.async_copy(mti_hbm.at[iti_v.at[c]], gti_v.at[c], gsem))

    for g in range(NGRP):
        sl = pl.ds(g * 16, 16)
        tt = t_v[sl]
        uu = u_v[sl]
        ii = i_v[sl]
        bias_v[sl] = (a_v[:] - r_v[sl]
                      + plsc.load_gather(bi_v, [ii])
                      + plsc.load_gather(bt_v, [tt])
                      + plsc.load_gather(th_v, [uu]) * pr_v[sl]
                      + plsc.load_gather(bu_v, [uu]))

    for cp in copies:
        cp.wait()

    acc = jnp.zeros((16,), jnp.float32)
    for g in range(NGRP):
        sl = pl.ds(g * 16, 16)
        c, o = g // 8, (g % 8) * 16
        osl = pl.ds(o, 16)
        diff = bias_v[sl] + gui_v[c, osl] + gij_v[c, osl] + gti_v[c, osl]
        acc = acc + diff * diff
    acc_v[:] = acc
    pltpu.sync_copy(acc_v, out_hbm.at[wid])


def kernel(sampleT, sampleU, sampleI, sampleJ, samplePR, sampleR, alpha,
           betaU, betaI, betaT, thetaU,
           gammaUI, gammaIU, gammaIJ, gammaJI, gammaTI, gammaIT):
    mui, mij, mti = _tables(gammaUI.T, gammaIU.T, gammaIJ.T, gammaJI.T,
                            gammaTI.T, gammaIT.T)
    alpha16 = jnp.full((16,), alpha, jnp.float32)
    out = _sc_loss(sampleT, sampleU, sampleI, sampleJ, samplePR, sampleR,
                   alpha16, betaU, betaI, betaT, thetaU,
                   mui.reshape(-1), mij.reshape(-1), mti.reshape(-1))
    return jnp.sum(out) * (0.5 / NB)


# speedup vs baseline: 22.6933x; 1.0112x over previous
import functools

import jax
import jax.numpy as jnp
from jax import lax
from jax.experimental import pallas as pl
from jax.experimental.pallas import tpu as pltpu
from jax.experimental.pallas import tpu_sc as plsc

NC, NS = 2, 16
NW = NC * NS
NB = 16384
BPW = NB // NW
NGRP = BPW // 16
NCHUNK = BPW // 128
N_UI = 1000
N_T = 100
TIW = 128


def _tables_body(ui_ref, iu_ref, ij_ref, ji_ref, ti_ref, it_ref,
                 mui_ref, mij_ref, mti_ref):
    dn = (((0,), (0,)), ((), ()))
    mui = lax.dot_general(ui_ref[...], iu_ref[...], dn,
                          preferred_element_type=jnp.float32)
    mij = lax.dot_general(ij_ref[...], ji_ref[...], dn,
                          preferred_element_type=jnp.float32)
    for k in range(4):
        mui_ref[:, k, 0, :] = mui[:, k * 128:(k + 1) * 128]
        mij_ref[:, k, 0, :] = mij[:, k * 128:(k + 1) * 128]

    @pl.when(pl.program_id(0) == 0)
    def _():
        it128 = jnp.concatenate(
            [it_ref[...], jnp.full((K, TIW - N_T), jnp.nan, jnp.float32)],
            axis=1)
        mti_ref[...] = lax.dot_general(ti_ref[...], it128, dn,
                                       preferred_element_type=jnp.float32)


K = 64
_tables = pl.pallas_call(
    _tables_body,
    grid=(2,),
    in_specs=[
        pl.BlockSpec((K, N_UI), lambda b: (0, 0)),
        pl.BlockSpec((K, 512), lambda b: (0, b)),
        pl.BlockSpec((K, N_UI), lambda b: (0, 0)),
        pl.BlockSpec((K, 512), lambda b: (0, b)),
        pl.BlockSpec((K, N_T), lambda b: (0, 0)),
        pl.BlockSpec((K, N_T), lambda b: (0, 0)),
    ],
    out_specs=[
        pl.BlockSpec((N_UI, 4, 1, 128), lambda b: (0, b, 0, 0)),
        pl.BlockSpec((N_UI, 4, 1, 128), lambda b: (0, b, 0, 0)),
        pl.BlockSpec((N_T, TIW), lambda b: (0, 0)),
    ],
    out_shape=[
        jax.ShapeDtypeStruct((N_UI, 8, 1, 128), jnp.float32),
        jax.ShapeDtypeStruct((N_UI, 8, 1, 128), jnp.float32),
        jax.ShapeDtypeStruct((N_T, TIW), jnp.float32),
    ],
)

_mesh = plsc.VectorSubcoreMesh(core_axis_name="c", subcore_axis_name="s",
                               num_cores=NC, num_subcores=NS)


@functools.partial(
    pl.kernel,
    out_type=jax.ShapeDtypeStruct((NW, 16), jnp.float32),
    mesh=_mesh,
    compiler_params=pltpu.CompilerParams(needs_layout_passes=False),
    scratch_types=[
        pltpu.VMEM((BPW,), jnp.int32),
        pltpu.VMEM((BPW,), jnp.int32),
        pltpu.VMEM((BPW,), jnp.int32),
        pltpu.VMEM((BPW,), jnp.int32),
        pltpu.VMEM((BPW,), jnp.float32),
        pltpu.VMEM((BPW,), jnp.float32),
        pltpu.VMEM((16,), jnp.float32),
        pltpu.VMEM((1024,), jnp.float32),
        pltpu.VMEM((1024,), jnp.float32),
        pltpu.VMEM((128,), jnp.float32),
        pltpu.VMEM((1024,), jnp.float32),
        pltpu.VMEM((NCHUNK, 128), jnp.int32),
        pltpu.VMEM((NCHUNK, 128), jnp.int32),
        pltpu.VMEM((NCHUNK, 128), jnp.int32),
        pltpu.VMEM((NCHUNK, 128), jnp.float32),
        pltpu.VMEM((NCHUNK, 128), jnp.float32),
        pltpu.VMEM((NCHUNK, 128), jnp.float32),
        pltpu.VMEM((BPW,), jnp.float32),
        pltpu.VMEM((16,), jnp.float32),
        pltpu.SemaphoreType.DMA,
        pltpu.SemaphoreType.DMA,
    ],
)
def _sc_loss(t_hbm, u_hbm, i_hbm, j_hbm, pr_hbm, r_hbm, a_hbm,
             bu_hbm, bi_hbm, bt_hbm, th_hbm,
             mui_hbm, mij_hbm, mti_hbm, out_hbm,
             t_v, u_v, i_v, j_v, pr_v, r_v, a_v,
             bu_v, bi_v, bt_v, th_v,
             iui_v, iij_v, iti_v, gui_v, gij_v, gti_v, bias_v, acc_v,
             sem, gsem):
    wid = lax.axis_index("s") * NC + lax.axis_index("c")
    base = wid * BPW

    stage = [
        pltpu.async_copy(t_hbm.at[pl.ds(base, BPW)], t_v, sem),
        pltpu.async_copy(u_hbm.at[pl.ds(base, BPW)], u_v, sem),
        pltpu.async_copy(i_hbm.at[pl.ds(base, BPW)], i_v, sem),
        pltpu.async_copy(j_hbm.at[pl.ds(base, BPW)], j_v, sem),
        pltpu.async_copy(pr_hbm.at[pl.ds(base, BPW)], pr_v, sem),
        pltpu.async_copy(r_hbm.at[pl.ds(base, BPW)], r_v, sem),
        pltpu.async_copy(a_hbm, a_v, sem),
        pltpu.async_copy(bu_hbm, bu_v.at[pl.ds(0, N_UI)], sem),
        pltpu.async_copy(bi_hbm, bi_v.at[pl.ds(0, N_UI)], sem),
        pltpu.async_copy(bt_hbm, bt_v.at[pl.ds(0, N_T)], sem),
        pltpu.async_copy(th_hbm, th_v.at[pl.ds(0, N_UI)], sem),
    ]
    for cp in stage:
        cp.wait()

    def _idx_body(g, _):
        sl = pl.ds(g * 16, 16)
        c = g // 8
        osl = pl.ds((g % 8) * 16, 16)
        tt = t_v[sl]
        uu = u_v[sl]
        ii = i_v[sl]
        jj = j_v[sl]
        iui_v[c, osl] = uu * 1024 + ii
        iij_v[c, osl] = ii * 1024 + jj
        iti_v[c, osl] = tt * TIW + jnp.minimum(ii, N_T)
        return 0

    lax.fori_loop(0, NGRP, _idx_body, 0)

    def _fire_body(c, _):
        pltpu.async_copy(mui_hbm.at[iui_v.at[c]], gui_v.at[c], gsem)
        pltpu.async_copy(mij_hbm.at[iij_v.at[c]], gij_v.at[c], gsem)
        pltpu.async_copy(mti_hbm.at[iti_v.at[c]], gti_v.at[c], gsem)
        return 0

    lax.fori_loop(0, NCHUNK, _fire_body, 0)

    def _bias_body(g, _):
        sl = pl.ds(g * 16, 16)
        tt = t_v[sl]
        uu = u_v[sl]
        ii = i_v[sl]
        bias_v[sl] = (a_v[:] - r_v[sl]
                      + plsc.load_gather(bi_v, [ii])
                      + plsc.load_gather(bt_v, [tt])
                      + plsc.load_gather(th_v, [uu]) * pr_v[sl]
                      + plsc.load_gather(bu_v, [uu]))
        return 0

    lax.fori_loop(0, NGRP, _bias_body, 0)

    def _drain_body(c, _):
        pltpu.make_async_copy(mui_hbm.at[iui_v.at[c]], gui_v.at[c], gsem).wait()
        pltpu.make_async_copy(mij_hbm.at[iij_v.at[c]], gij_v.at[c], gsem).wait()
        pltpu.make_async_copy(mti_hbm.at[iti_v.at[c]], gti_v.at[c], gsem).wait()
        return 0

    lax.fori_loop(0, NCHUNK, _drain_body, 0)

    def _acc_body(g, acc):
        sl = pl.ds(g * 16, 16)
        c = g // 8
        osl = pl.ds((g % 8) * 16, 16)
        diff = bias_v[sl] + gui_v[c, osl] + gij_v[c, osl] + gti_v[c, osl]
        return acc + diff * diff

    acc_v[:] = lax.fori_loop(0, NGRP, _acc_body, jnp.zeros((16,), jnp.float32))
    pltpu.sync_copy(acc_v, out_hbm.at[wid])


def kernel(sampleT, sampleU, sampleI, sampleJ, samplePR, sampleR, alpha,
           betaU, betaI, betaT, thetaU,
           gammaUI, gammaIU, gammaIJ, gammaJI, gammaTI, gammaIT):
    mui, mij, mti = _tables(gammaUI.T, gammaIU.T, gammaIJ.T, gammaJI.T,
                            gammaTI.T, gammaIT.T)
    alpha16 = jnp.full((16,), alpha, jnp.float32)
    out = _sc_loss(sampleT, sampleU, sampleI, sampleJ, samplePR, sampleR,
                   alpha16, betaU, betaI, betaT, thetaU,
                   mui.reshape(-1), mij.reshape(-1), mti.reshape(-1))
    return jnp.sum(out) * (0.5 / NB)


# speedup vs baseline: 22.8461x; 1.0067x over previous
import functools

import jax
import jax.numpy as jnp
from jax import lax
from jax.experimental import pallas as pl
from jax.experimental.pallas import tpu as pltpu
from jax.experimental.pallas import tpu_sc as plsc

NC, NS = 2, 16
NW = NC * NS
NB = 16384
BPW = NB // NW
NGRP = BPW // 16
N_UI = 1000
N_T = 100
TIW = 128
K = 64
MROWS = 2 * N_UI + N_T
NIDX = 3 * BPW // 128


def _tables_body(ui_ref, iu_ref, ij_ref, ji_ref, ti_ref, it_ref, m_ref):
    dn = (((0,), (0,)), ((), ()))
    mui = lax.dot_general(ui_ref[...], iu_ref[...], dn,
                          preferred_element_type=jnp.float32)
    mij = lax.dot_general(ij_ref[...], ji_ref[...], dn,
                          preferred_element_type=jnp.float32)
    for k in range(4):
        m_ref[0:N_UI, k, 0, :] = mui[:, k * 128:(k + 1) * 128]
        m_ref[N_UI:2 * N_UI, k, 0, :] = mij[:, k * 128:(k + 1) * 128]

    @pl.when(pl.program_id(0) == 0)
    def _():
        it128 = jnp.concatenate(
            [it_ref[...], jnp.full((K, TIW - N_T), jnp.nan, jnp.float32)],
            axis=1)
        m_ref[2 * N_UI:MROWS, 0, 0, :] = lax.dot_general(
            ti_ref[...], it128, dn, preferred_element_type=jnp.float32)


_tables = pl.pallas_call(
    _tables_body,
    grid=(2,),
    in_specs=[
        pl.BlockSpec((K, N_UI), lambda b: (0, 0)),
        pl.BlockSpec((K, 512), lambda b: (0, b)),
        pl.BlockSpec((K, N_UI), lambda b: (0, 0)),
        pl.BlockSpec((K, 512), lambda b: (0, b)),
        pl.BlockSpec((K, N_T), lambda b: (0, 0)),
        pl.BlockSpec((K, N_T), lambda b: (0, 0)),
    ],
    out_specs=pl.BlockSpec((MROWS, 4, 1, 128), lambda b: (0, b, 0, 0)),
    out_shape=jax.ShapeDtypeStruct((MROWS, 8, 1, 128), jnp.float32),
)

_mesh = plsc.VectorSubcoreMesh(core_axis_name="c", subcore_axis_name="s",
                               num_cores=NC, num_subcores=NS)


@functools.partial(
    pl.kernel,
    out_type=jax.ShapeDtypeStruct((NW, 16), jnp.float32),
    mesh=_mesh,
    compiler_params=pltpu.CompilerParams(needs_layout_passes=False),
    scratch_types=[
        pltpu.VMEM((BPW,), jnp.int32),
        pltpu.VMEM((BPW,), jnp.int32),
        pltpu.VMEM((BPW,), jnp.int32),
        pltpu.VMEM((BPW,), jnp.int32),
        pltpu.VMEM((BPW,), jnp.float32),
        pltpu.VMEM((BPW,), jnp.float32),
        pltpu.VMEM((16,), jnp.float32),
        pltpu.VMEM((1024,), jnp.float32),
        pltpu.VMEM((1024,), jnp.float32),
        pltpu.VMEM((128,), jnp.float32),
        pltpu.VMEM((1024,), jnp.float32),
        pltpu.VMEM((3 * BPW,), jnp.int32),
        pltpu.VMEM((3 * BPW,), jnp.float32),
        pltpu.VMEM((BPW,), jnp.float32),
        pltpu.VMEM((16,), jnp.float32),
        pltpu.SemaphoreType.DMA,
        pltpu.SemaphoreType.DMA,
        pltpu.SemaphoreType.DMA,
    ],
)
def _sc_loss(t_hbm, u_hbm, i_hbm, j_hbm, pr_hbm, r_hbm, a_hbm,
             bu_hbm, bi_hbm, bt_hbm, th_hbm, m_hbm, out_hbm,
             t_v, u_v, i_v, j_v, pr_v, r_v, a_v,
             bu_v, bi_v, bt_v, th_v,
             idx_v, g_v, bias_v, acc_v, sem, sem2, gsem):
    wid = lax.axis_index("s") * NC + lax.axis_index("c")
    base = wid * BPW

    stage1 = [
        pltpu.async_copy(t_hbm.at[pl.ds(base, BPW)], t_v, sem),
        pltpu.async_copy(u_hbm.at[pl.ds(base, BPW)], u_v, sem),
        pltpu.async_copy(i_hbm.at[pl.ds(base, BPW)], i_v, sem),
        pltpu.async_copy(j_hbm.at[pl.ds(base, BPW)], j_v, sem),
    ]
    stage2 = [
        pltpu.async_copy(pr_hbm.at[pl.ds(base, BPW)], pr_v, sem2),
        pltpu.async_copy(r_hbm.at[pl.ds(base, BPW)], r_v, sem2),
        pltpu.async_copy(a_hbm, a_v, sem2),
        pltpu.async_copy(bu_hbm, bu_v.at[pl.ds(0, N_UI)], sem2),
        pltpu.async_copy(bi_hbm, bi_v.at[pl.ds(0, N_UI)], sem2),
        pltpu.async_copy(bt_hbm, bt_v.at[pl.ds(0, N_T)], sem2),
        pltpu.async_copy(th_hbm, th_v.at[pl.ds(0, N_UI)], sem2),
    ]
    for cp in stage1:
        cp.wait()

    def _idx_body(g, _):
        sl = pl.ds(g * 16, 16)
        tt = t_v[sl]
        uu = u_v[sl]
        ii = i_v[sl]
        jj = j_v[sl]
        idx_v[sl] = uu * 1024 + ii
        idx_v[pl.ds(BPW + g * 16, 16)] = (N_UI + ii) * 1024 + jj
        idx_v[pl.ds(2 * BPW + g * 16, 16)] = (
            (2 * N_UI + tt) * 1024 + jnp.minimum(ii, N_T))
        return 0

    lax.fori_loop(0, NGRP, _idx_body, 0)

    gather = pltpu.async_copy(m_hbm.at[idx_v], g_v, gsem)

    for cp in stage2:
        cp.wait()

    def _bias_body(g, _):
        sl = pl.ds(g * 16, 16)
        tt = t_v[sl]
        uu = u_v[sl]
        ii = i_v[sl]
        bias_v[sl] = (a_v[:] - r_v[sl]
                      + plsc.load_gather(bi_v, [ii])
                      + plsc.load_gather(bt_v, [tt])
                      + plsc.load_gather(th_v, [uu]) * pr_v[sl]
                      + plsc.load_gather(bu_v, [uu]))
        return 0

    lax.fori_loop(0, NGRP, _bias_body, 0)

    gather.wait()

    def _acc_body(g, acc):
        sl = pl.ds(g * 16, 16)
        diff = (bias_v[sl] + g_v[sl] + g_v[pl.ds(BPW + g * 16, 16)]
                + g_v[pl.ds(2 * BPW + g * 16, 16)])
        return acc + diff * diff

    acc_v[:] = lax.fori_loop(0, NGRP, _acc_body, jnp.zeros((16,), jnp.float32))
    pltpu.sync_copy(acc_v, out_hbm.at[wid])


def kernel(sampleT, sampleU, sampleI, sampleJ, samplePR, sampleR, alpha,
           betaU, betaI, betaT, thetaU,
           gammaUI, gammaIU, gammaIJ, gammaJI, gammaTI, gammaIT):
    m = _tables(gammaUI.T, gammaIU.T, gammaIJ.T, gammaJI.T,
                gammaTI.T, gammaIT.T)
    alpha16 = jnp.full((16,), alpha, jnp.float32)
    out = _sc_loss(sampleT, sampleU, sampleI, sampleJ, samplePR, sampleR,
                   alpha16, betaU, betaI, betaT, thetaU, m.reshape(-1))
    return jnp.sum(out) * (0.5 / NB)


# speedup vs baseline: 23.2963x; 1.0197x over previous
import functools

import jax
import jax.numpy as jnp
from jax import lax
from jax.experimental import pallas as pl
from jax.experimental.pallas import tpu as pltpu
from jax.experimental.pallas import tpu_sc as plsc

NC, NS = 2, 16
NW = NC * NS
NB = 16384
BPW = NB // NW
NGRP = BPW // 16
N_UI = 1000
N_T = 100
TIW = 128
K = 64
MROWS = 2 * N_UI + N_T
NIDX = 3 * BPW // 128


def _tables_body(ui_ref, iu_ref, ij_ref, ji_ref, ti_ref, it_ref, m_ref):
    dn = (((0,), (0,)), ((), ()))
    mui = lax.dot_general(ui_ref[...], iu_ref[...], dn,
                          preferred_element_type=jnp.float32)
    mij = lax.dot_general(ij_ref[...], ji_ref[...], dn,
                          preferred_element_type=jnp.float32)
    for k in range(4):
        m_ref[k, 0:N_UI, 0, :] = mui[:, k * 128:(k + 1) * 128]
        m_ref[k, N_UI:2 * N_UI, 0, :] = mij[:, k * 128:(k + 1) * 128]

    @pl.when(pl.program_id(0) == 0)
    def _():
        it128 = jnp.concatenate(
            [it_ref[...], jnp.full((K, TIW - N_T), jnp.nan, jnp.float32)],
            axis=1)
        m_ref[0, 2 * N_UI:MROWS, 0, :] = lax.dot_general(
            ti_ref[...], it128, dn, preferred_element_type=jnp.float32)


_tables = pl.pallas_call(
    _tables_body,
    grid=(2,),
    in_specs=[
        pl.BlockSpec((K, N_UI), lambda b: (0, 0)),
        pl.BlockSpec((K, 512), lambda b: (0, b)),
        pl.BlockSpec((K, N_UI), lambda b: (0, 0)),
        pl.BlockSpec((K, 512), lambda b: (0, b)),
        pl.BlockSpec((K, N_T), lambda b: (0, 0)),
        pl.BlockSpec((K, N_T), lambda b: (0, 0)),
    ],
    out_specs=pl.BlockSpec((4, MROWS, 1, 128), lambda b: (b, 0, 0, 0)),
    out_shape=jax.ShapeDtypeStruct((8, MROWS, 1, 128), jnp.float32),
)

MBLK = MROWS * 128

_mesh = plsc.VectorSubcoreMesh(core_axis_name="c", subcore_axis_name="s",
                               num_cores=NC, num_subcores=NS)


@functools.partial(
    pl.kernel,
    out_type=jax.ShapeDtypeStruct((NW, 16), jnp.float32),
    mesh=_mesh,
    compiler_params=pltpu.CompilerParams(needs_layout_passes=False),
    scratch_types=[
        pltpu.VMEM((BPW,), jnp.int32),
        pltpu.VMEM((BPW,), jnp.int32),
        pltpu.VMEM((BPW,), jnp.int32),
        pltpu.VMEM((BPW,), jnp.int32),
        pltpu.VMEM((BPW,), jnp.float32),
        pltpu.VMEM((BPW,), jnp.float32),
        pltpu.VMEM((16,), jnp.float32),
        pltpu.VMEM((1024,), jnp.float32),
        pltpu.VMEM((1024,), jnp.float32),
        pltpu.VMEM((128,), jnp.float32),
        pltpu.VMEM((1024,), jnp.float32),
        pltpu.VMEM((3 * BPW,), jnp.int32),
        pltpu.VMEM((3 * BPW,), jnp.float32),
        pltpu.VMEM((BPW,), jnp.float32),
        pltpu.VMEM((16,), jnp.float32),
        pltpu.SemaphoreType.DMA,
        pltpu.SemaphoreType.DMA,
        pltpu.SemaphoreType.DMA,
    ],
)
def _sc_loss(t_hbm, u_hbm, i_hbm, j_hbm, pr_hbm, r_hbm, a_hbm,
             bu_hbm, bi_hbm, bt_hbm, th_hbm, m_hbm, out_hbm,
             t_v, u_v, i_v, j_v, pr_v, r_v, a_v,
             bu_v, bi_v, bt_v, th_v,
             idx_v, g_v, bias_v, acc_v, sem, sem2, gsem):
    wid = lax.axis_index("s") * NC + lax.axis_index("c")
    base = wid * BPW

    stage1 = [
        pltpu.async_copy(t_hbm.at[pl.ds(base, BPW)], t_v, sem),
        pltpu.async_copy(u_hbm.at[pl.ds(base, BPW)], u_v, sem),
        pltpu.async_copy(i_hbm.at[pl.ds(base, BPW)], i_v, sem),
        pltpu.async_copy(j_hbm.at[pl.ds(base, BPW)], j_v, sem),
    ]
    stage2 = [
        pltpu.async_copy(pr_hbm.at[pl.ds(base, BPW)], pr_v, sem2),
        pltpu.async_copy(r_hbm.at[pl.ds(base, BPW)], r_v, sem2),
        pltpu.async_copy(a_hbm, a_v, sem2),
        pltpu.async_copy(bu_hbm, bu_v.at[pl.ds(0, N_UI)], sem2),
        pltpu.async_copy(bi_hbm, bi_v.at[pl.ds(0, N_UI)], sem2),
        pltpu.async_copy(bt_hbm, bt_v.at[pl.ds(0, N_T)], sem2),
        pltpu.async_copy(th_hbm, th_v.at[pl.ds(0, N_UI)], sem2),
    ]
    for cp in stage1:
        cp.wait()

    def _idx_body(g, _):
        sl = pl.ds(g * 16, 16)
        tt = t_v[sl]
        uu = u_v[sl]
        ii = i_v[sl]
        jj = j_v[sl]
        idx_v[sl] = (ii >> 7) * MBLK + uu * 128 + (ii & 127)
        idx_v[pl.ds(BPW + g * 16, 16)] = (
            (jj >> 7) * MBLK + (N_UI + ii) * 128 + (jj & 127))
        idx_v[pl.ds(2 * BPW + g * 16, 16)] = (
            (2 * N_UI + tt) * 128 + jnp.minimum(ii, N_T))
        return 0

    lax.fori_loop(0, NGRP, _idx_body, 0)

    gather = pltpu.async_copy(m_hbm.at[idx_v], g_v, gsem)

    for cp in stage2:
        cp.wait()

    def _bias_body(g, _):
        sl = pl.ds(g * 16, 16)
        tt = t_v[sl]
        uu = u_v[sl]
        ii = i_v[sl]
        bias_v[sl] = (a_v[:] - r_v[sl]
                      + plsc.load_gather(bi_v, [ii])
                      + plsc.load_gather(bt_v, [tt])
                      + plsc.load_gather(th_v, [uu]) * pr_v[sl]
                      + plsc.load_gather(bu_v, [uu]))
        return 0

    lax.fori_loop(0, NGRP, _bias_body, 0)

    gather.wait()

    def _acc_body(g, acc):
        sl = pl.ds(g * 16, 16)
        diff = (bias_v[sl] + g_v[sl] + g_v[pl.ds(BPW + g * 16, 16)]
                + g_v[pl.ds(2 * BPW + g * 16, 16)])
        return acc + diff * diff

    acc_v[:] = lax.fori_loop(0, NGRP, _acc_body, jnp.zeros((16,), jnp.float32))
    pltpu.sync_copy(acc_v, out_hbm.at[wid])


def kernel(sampleT, sampleU, sampleI, sampleJ, samplePR, sampleR, alpha,
           betaU, betaI, betaT, thetaU,
           gammaUI, gammaIU, gammaIJ, gammaJI, gammaTI, gammaIT):
    m = _tables(gammaUI.T, gammaIU.T, gammaIJ.T, gammaJI.T,
                gammaTI.T, gammaIT.T)
    alpha16 = jnp.full((16,), alpha, jnp.float32)
    out = _sc_loss(sampleT, sampleU, sampleI, sampleJ, samplePR, sampleR,
                   alpha16, betaU, betaI, betaT, thetaU, m.reshape(-1))
    return jnp.sum(out) * (0.5 / NB)


# speedup vs baseline: 23.3889x; 1.0040x over previous
import functools

import jax
import jax.numpy as jnp
from jax import lax
from jax.experimental import pallas as pl
from jax.experimental.pallas import tpu as pltpu
from jax.experimental.pallas import tpu_sc as plsc

NC, NS = 2, 16
NW = NC * NS
NB = 16384
BPW = NB // NW
NGRP = BPW // 16
N_UI = 1000
N_T = 100
TIW = 128
K = 64
MROWS = 2 * N_UI + N_T
NIDX = 3 * BPW // 128


def _tables_body(ui_ref, iu_ref, ij_ref, ji_ref, ti_ref, it_ref, m_ref):
    dn = (((0,), (0,)), ((), ()))
    mui = lax.dot_general(ui_ref[...], iu_ref[...], dn,
                          preferred_element_type=jnp.float32)
    mij = lax.dot_general(ij_ref[...], ji_ref[...], dn,
                          preferred_element_type=jnp.float32)
    for k in range(2):
        m_ref[k, 0:N_UI, 0, :] = mui[:, k * 128:(k + 1) * 128]
        m_ref[k, N_UI:2 * N_UI, 0, :] = mij[:, k * 128:(k + 1) * 128]

    @pl.when(pl.program_id(0) == 0)
    def _():
        it128 = jnp.concatenate(
            [it_ref[...], jnp.full((K, TIW - N_T), jnp.nan, jnp.float32)],
            axis=1)
        m_ref[0, 2 * N_UI:MROWS, 0, :] = lax.dot_general(
            ti_ref[...], it128, dn, preferred_element_type=jnp.float32)


_tables = pl.pallas_call(
    _tables_body,
    grid=(4,),
    in_specs=[
        pl.BlockSpec((K, N_UI), lambda b: (0, 0)),
        pl.BlockSpec((K, 256), lambda b: (0, b)),
        pl.BlockSpec((K, N_UI), lambda b: (0, 0)),
        pl.BlockSpec((K, 256), lambda b: (0, b)),
        pl.BlockSpec((K, N_T), lambda b: (0, 0)),
        pl.BlockSpec((K, N_T), lambda b: (0, 0)),
    ],
    out_specs=pl.BlockSpec((2, MROWS, 1, 128), lambda b: (b, 0, 0, 0)),
    out_shape=jax.ShapeDtypeStruct((8, MROWS, 1, 128), jnp.float32),
)

MBLK = MROWS * 128

_mesh = plsc.VectorSubcoreMesh(core_axis_name="c", subcore_axis_name="s",
                               num_cores=NC, num_subcores=NS)


@functools.partial(
    pl.kernel,
    out_type=jax.ShapeDtypeStruct((NW, 16), jnp.float32),
    mesh=_mesh,
    compiler_params=pltpu.CompilerParams(needs_layout_passes=False),
    scratch_types=[
        pltpu.VMEM((BPW,), jnp.int32),
        pltpu.VMEM((BPW,), jnp.int32),
        pltpu.VMEM((BPW,), jnp.int32),
        pltpu.VMEM((BPW,), jnp.int32),
        pltpu.VMEM((BPW,), jnp.float32),
        pltpu.VMEM((BPW,), jnp.float32),
        pltpu.VMEM((16,), jnp.float32),
        pltpu.VMEM((1024,), jnp.float32),
        pltpu.VMEM((1024,), jnp.float32),
        pltpu.VMEM((128,), jnp.float32),
        pltpu.VMEM((1024,), jnp.float32),
        pltpu.VMEM((3 * BPW,), jnp.int32),
        pltpu.VMEM((3 * BPW,), jnp.float32),
        pltpu.VMEM((BPW,), jnp.float32),
        pltpu.VMEM((16,), jnp.float32),
        pltpu.SemaphoreType.DMA,
        pltpu.SemaphoreType.DMA,
        pltpu.SemaphoreType.DMA,
    ],
)
def _sc_loss(t_hbm, u_hbm, i_hbm, j_hbm, pr_hbm, r_hbm, a_hbm,
             bu_hbm, bi_hbm, bt_hbm, th_hbm, m_hbm, out_hbm,
             t_v, u_v, i_v, j_v, pr_v, r_v, a_v,
             bu_v, bi_v, bt_v, th_v,
             idx_v, g_v, bias_v, acc_v, sem, sem2, gsem):
    wid = lax.axis_index("s") * NC + lax.axis_index("c")
    base = wid * BPW

    stage1 = [
        pltpu.async_copy(t_hbm.at[pl.ds(base, BPW)], t_v, sem),
        pltpu.async_copy(u_hbm.at[pl.ds(base, BPW)], u_v, sem),
        pltpu.async_copy(i_hbm.at[pl.ds(base, BPW)], i_v, sem),
        pltpu.async_copy(j_hbm.at[pl.ds(base, BPW)], j_v, sem),
    ]
    stage2 = [
        pltpu.async_copy(pr_hbm.at[pl.ds(base, BPW)], pr_v, sem2),
        pltpu.async_copy(r_hbm.at[pl.ds(base, BPW)], r_v, sem2),
        pltpu.async_copy(a_hbm, a_v, sem2),
        pltpu.async_copy(bu_hbm, bu_v.at[pl.ds(0, N_UI)], sem2),
        pltpu.async_copy(bi_hbm, bi_v.at[pl.ds(0, N_UI)], sem2),
        pltpu.async_copy(bt_hbm, bt_v.at[pl.ds(0, N_T)], sem2),
        pltpu.async_copy(th_hbm, th_v.at[pl.ds(0, N_UI)], sem2),
    ]
    for cp in stage1:
        cp.wait()

    def _idx_body(g, _):
        sl = pl.ds(g * 16, 16)
        tt = t_v[sl]
        uu = u_v[sl]
        ii = i_v[sl]
        jj = j_v[sl]
        idx_v[sl] = (ii >> 7) * MBLK + uu * 128 + (ii & 127)
        idx_v[pl.ds(BPW + g * 16, 16)] = (
            (jj >> 7) * MBLK + (N_UI + ii) * 128 + (jj & 127))
        idx_v[pl.ds(2 * BPW + g * 16, 16)] = (
            (2 * N_UI + tt) * 128 + jnp.minimum(ii, N_T))
        return 0

    lax.fori_loop(0, NGRP, _idx_body, 0)

    gather = pltpu.async_copy(m_hbm.at[idx_v], g_v, gsem)

    for cp in stage2:
        cp.wait()

    def _bias_body(g, _):
        sl = pl.ds(g * 16, 16)
        tt = t_v[sl]
        uu = u_v[sl]
        ii = i_v[sl]
        bias_v[sl] = (a_v[:] - r_v[sl]
                      + plsc.load_gather(bi_v, [ii])
                      + plsc.load_gather(bt_v, [tt])
                      + plsc.load_gather(th_v, [uu]) * pr_v[sl]
                      + plsc.load_gather(bu_v, [uu]))
        return 0

    lax.fori_loop(0, NGRP, _bias_body, 0)

    gather.wait()

    def _acc_body(g, acc):
        sl = pl.ds(g * 16, 16)
        diff = (bias_v[sl] + g_v[sl] + g_v[pl.ds(BPW + g * 16, 16)]
                + g_v[pl.ds(2 * BPW + g * 16, 16)])
        return acc + diff * diff

    acc_v[:] = lax.fori_loop(0, NGRP, _acc_body, jnp.zeros((16,), jnp.float32))
    pltpu.sync_copy(acc_v, out_hbm.at[wid])


def kernel(sampleT, sampleU, sampleI, sampleJ, samplePR, sampleR, alpha,
           betaU, betaI, betaT, thetaU,
           gammaUI, gammaIU, gammaIJ, gammaJI, gammaTI, gammaIT):
    m = _tables(gammaUI.T, gammaIU.T, gammaIJ.T, gammaJI.T,
                gammaTI.T, gammaIT.T)
    alpha16 = jnp.full((16,), alpha, jnp.float32)
    out = _sc_loss(sampleT, sampleU, sampleI, sampleJ, samplePR, sampleR,
                   alpha16, betaU, betaI, betaT, thetaU, m.reshape(-1))
    return jnp.sum(out) * (0.5 / NB)
